# baseline (speedup 1.0000x reference)
import functools

import numpy as np
import jax
import jax.numpy as jnp
from jax import lax
from jax.experimental import pallas as pl
from jax.experimental.pallas import tpu as pltpu
from jax.experimental.pallas import tpu_sc as plsc

_NUM_USERS = 100000
_NUM_ITEMS = 100000
_POOL_SIZE = 200
_NUM_NEG = 50
_BATCH = 16384

_NC, _NS, _L = 2, 16, 16
_NW = _NC * _NS
_ROWS_PER_W = _BATCH // _NW
_G = 128
_NGROUPS = _ROWS_PER_W // _G
_EPG = _G * _NUM_NEG
_VECS = _EPG // _L

_SEL_ROW = np.repeat((np.arange(_BATCH, dtype=np.int32) % _G), _NUM_NEG)

_MESH = plsc.VectorSubcoreMesh(
    core_axis_name="c", subcore_axis_name="s",
    num_cores=_NC, num_subcores=_NS)


_KERNEL_CFG = dict(
    out_type=jax.ShapeDtypeStruct((_BATCH * _NUM_NEG,), jnp.int32),
    mesh=_MESH,
    compiler_params=pltpu.CompilerParams(use_tc_tiling_on_sc=False,
                                         needs_layout_passes=False),
    scratch_types=[
        pltpu.VMEM((_G,), jnp.int32),
        pltpu.VMEM((_G, _POOL_SIZE), jnp.int32),
        pltpu.VMEM((_EPG,), jnp.int32),
        pltpu.VMEM((_EPG,), jnp.int32),
        pltpu.VMEM((_EPG,), jnp.int32),
        pltpu.SemaphoreType.DMA,
    ],
)


def _neg_gather_body(user_id_hbm, selrow_hbm, selcol_hbm, pool_hbm, out_hbm,
                uid_v, rows_v, selrow_v, selcol_v, out_v, sem):
    wid = lax.axis_index("s") * _NC + lax.axis_index("c")
    row0 = wid * _ROWS_PER_W

    def group(g, carry):
        rbase = row0 + g * _G
        ebase = rbase * _NUM_NEG
        pltpu.sync_copy(user_id_hbm.at[pl.ds(rbase, _G)], uid_v)
        cp = pltpu.async_copy(pool_hbm.at[uid_v], rows_v, sem)
        pltpu.sync_copy(selrow_hbm.at[pl.ds(ebase, _EPG)], selrow_v)
        pltpu.sync_copy(selcol_hbm.at[pl.ds(ebase, _EPG)], selcol_v)
        cp.wait()

        def vec(i, c2):
            sl = pl.ds(i * _L, _L)
            r = selrow_v[sl]
            col = selcol_v[sl]
            out_v[sl] = plsc.load_gather(rows_v, [r, col])
            return c2

        lax.fori_loop(0, _VECS, vec, 0)
        pltpu.sync_copy(out_v, out_hbm.at[pl.ds(ebase, _EPG)])
        return carry

    lax.fori_loop(0, _NGROUPS, group, 0)


_neg_gather = pl.kernel(_neg_gather_body, **_KERNEL_CFG)


def kernel(user_id, pool):
    idx_k = jax.random.randint(jax.random.key(42), (_BATCH, _NUM_NEG), 0,
                               _POOL_SIZE, dtype=jnp.int32)
    neg_flat = _neg_gather(user_id, jnp.asarray(_SEL_ROW),
                           idx_k.reshape(-1), pool)
    neg_items = neg_flat.reshape(_BATCH, _NUM_NEG)
    log_q = jnp.full((_BATCH, _NUM_NEG), -np.log(float(_POOL_SIZE)),
                     dtype=jnp.float32)
    return (neg_items, log_q)
